# baseline (speedup 1.0000x reference)
import functools

import jax
import jax.numpy as jnp
from jax import lax
from jax.experimental import pallas as pl
from jax.experimental.pallas import tpu as pltpu
from jax.experimental.pallas import tpu_sc as plsc

NUM_CLASSES = 1000
N_CTX = 16
CTX_DIM = 768
SEQ_LEN = 77
BATCH = 1024

N_SUF = SEQ_LEN - 1 - N_CTX

_info = plsc.get_sparse_core_info()
NC, NS, NL = _info.num_cores, _info.num_subcores, _info.num_lanes
NW = NC * NS
BPW = BATCH // NW

_mesh = plsc.VectorSubcoreMesh(core_axis_name="c", subcore_axis_name="s")


@functools.partial(
    pl.kernel,
    out_type=(
        jax.ShapeDtypeStruct((BATCH, N_CTX, CTX_DIM), jnp.float32),
        jax.ShapeDtypeStruct((BATCH, N_SUF, CTX_DIM), jnp.float32),
    ),
    mesh=_mesh,
    compiler_params=pltpu.CompilerParams(
        use_tc_tiling_on_sc=True, needs_layout_passes=False),
    scratch_types=[
        pltpu.VMEM((BPW,), jnp.int32),
        pltpu.VMEM((N_CTX, CTX_DIM), jnp.float32),
        pltpu.VMEM((N_CTX, CTX_DIM), jnp.float32),
        pltpu.VMEM((N_SUF, CTX_DIM), jnp.float32),
        pltpu.VMEM((N_SUF, CTX_DIM), jnp.float32),
        pltpu.SemaphoreType.DMA,
        pltpu.SemaphoreType.DMA,
        pltpu.SemaphoreType.DMA,
        pltpu.SemaphoreType.DMA,
        pltpu.SemaphoreType.DMA,
        pltpu.SemaphoreType.DMA,
        pltpu.SemaphoreType.DMA,
        pltpu.SemaphoreType.DMA,
    ],
)
def _gather_sc(label_hbm, ctx_hbm, suf_hbm, gctx_hbm, gsuf_hbm,
               idx_v, cbuf0, cbuf1, sbuf0, sbuf1,
               cg0, cg1, sg0, sg1, co0, co1, so0, so1):
    wid = lax.axis_index("s") * NC + lax.axis_index("c")
    base = wid * BPW

    pltpu.sync_copy(label_hbm.at[pl.ds(base, BPW)], idx_v)

    lanes = lax.iota(jnp.int32, NL)
    chunks = [idx_v[pl.ds(k * NL, NL)] for k in range(BPW // NL)]

    def label_of(e):
        v = jnp.where(lanes == (e % NL), chunks[e // NL], 0)
        return lax.reduce_max(v, (0,))

    cbufs = (cbuf0, cbuf1)
    sbufs = (sbuf0, sbuf1)
    cgs = (cg0, cg1)
    sgs = (sg0, sg1)
    cos = (co0, co1)
    sos = (so0, so1)

    def cin(e, s):
        return pltpu.make_async_copy(
            ctx_hbm.at[label_of(e)], cbufs[s], cgs[s])

    def sin(e, s):
        return pltpu.make_async_copy(
            suf_hbm.at[label_of(e)], sbufs[s], sgs[s])

    def cout(e, s):
        return pltpu.make_async_copy(cbufs[s], gctx_hbm.at[base + e], cos[s])

    def sout(e, s):
        return pltpu.make_async_copy(
            sbufs[s], gsuf_hbm.at[base + e], sos[s])

    for e in (0, 1):
        cin(e, e).start()
        sin(e, e).start()

    for e in range(BPW):
        s = e % 2
        cin(e, s).wait()
        cout(e, s).start()
        sin(e, s).wait()
        sout(e, s).start()
        if e + 2 < BPW:
            cout(e, s).wait()
            cin(e + 2, s).start()
            sout(e, s).wait()
            sin(e + 2, s).start()
        else:
            cout(e, s).wait()
            sout(e, s).wait()


B_BLK = 16


def _concat_tc(pre_ref, gctx_ref, gsuf_ref, out_ref):
    out_ref[:, 0:1, :] = jnp.broadcast_to(pre_ref[...], (B_BLK, 1, CTX_DIM))
    out_ref[:, 1:1 + N_CTX, :] = gctx_ref[...]
    out_ref[:, 1 + N_CTX:SEQ_LEN, :] = gsuf_ref[...]


_assemble_tc = pl.pallas_call(
    _concat_tc,
    out_shape=jax.ShapeDtypeStruct((BATCH, SEQ_LEN, CTX_DIM), jnp.float32),
    grid=(BATCH // B_BLK,),
    in_specs=[
        pl.BlockSpec((1, 1, CTX_DIM), lambda i: (0, 0, 0)),
        pl.BlockSpec((B_BLK, N_CTX, CTX_DIM), lambda i: (i, 0, 0)),
        pl.BlockSpec((B_BLK, N_SUF, CTX_DIM), lambda i: (i, 0, 0)),
    ],
    out_specs=pl.BlockSpec((B_BLK, SEQ_LEN, CTX_DIM), lambda i: (i, 0, 0)),
)


@jax.jit
def kernel(label, cls_ctx, token_prefix, token_suffix):
    gctx, gsuf = _gather_sc(label.astype(jnp.int32), cls_ctx, token_suffix)
    return _assemble_tc(token_prefix, gctx, gsuf)


# speedup vs baseline: 1.0002x; 1.0002x over previous
import functools

import jax
import jax.numpy as jnp
from jax import lax
from jax.experimental import pallas as pl
from jax.experimental.pallas import tpu as pltpu
from jax.experimental.pallas import tpu_sc as plsc

NUM_CLASSES = 1000
N_CTX = 16
CTX_DIM = 768
SEQ_LEN = 77
BATCH = 1024

N_SUF = SEQ_LEN - 1 - N_CTX

_info = plsc.get_sparse_core_info()
NC, NS, NL = _info.num_cores, _info.num_subcores, _info.num_lanes
NW = NC * NS
BPW = BATCH // NW

_mesh = plsc.VectorSubcoreMesh(core_axis_name="c", subcore_axis_name="s")


@functools.partial(
    pl.kernel,
    out_type=(
        jax.ShapeDtypeStruct((BATCH, N_CTX, CTX_DIM), jnp.float32),
        jax.ShapeDtypeStruct((BATCH, N_SUF, CTX_DIM), jnp.float32),
    ),
    mesh=_mesh,
    compiler_params=pltpu.CompilerParams(
        use_tc_tiling_on_sc=True, needs_layout_passes=False),
    scratch_types=[
        pltpu.VMEM((BPW,), jnp.int32),
        pltpu.VMEM((N_CTX, CTX_DIM), jnp.float32),
        pltpu.VMEM((N_CTX, CTX_DIM), jnp.float32),
        pltpu.VMEM((N_SUF, CTX_DIM), jnp.float32),
        pltpu.VMEM((N_SUF, CTX_DIM), jnp.float32),
        pltpu.SemaphoreType.DMA,
        pltpu.SemaphoreType.DMA,
        pltpu.SemaphoreType.DMA,
        pltpu.SemaphoreType.DMA,
        pltpu.SemaphoreType.DMA,
        pltpu.SemaphoreType.DMA,
        pltpu.SemaphoreType.DMA,
        pltpu.SemaphoreType.DMA,
    ],
)
def _gather_sc(label_hbm, ctx_hbm, suf_hbm, gctx_hbm, gsuf_hbm,
               idx_v, cbuf0, cbuf1, sbuf0, sbuf1,
               cg0, cg1, sg0, sg1, co0, co1, so0, so1):
    wid = lax.axis_index("s") * NC + lax.axis_index("c")
    base = wid * BPW

    pltpu.sync_copy(label_hbm.at[pl.ds(base, BPW)], idx_v)

    lanes = lax.iota(jnp.int32, NL)
    chunks = [idx_v[pl.ds(k * NL, NL)] for k in range(BPW // NL)]

    def label_of(e):
        v = jnp.where(lanes == (e % NL), chunks[e // NL], 0)
        return lax.reduce_max(v, (0,))

    labels = [label_of(e) for e in range(BPW)]

    cbufs = (cbuf0, cbuf1)
    sbufs = (sbuf0, sbuf1)
    cgs = (cg0, cg1)
    sgs = (sg0, sg1)
    cos = (co0, co1)
    sos = (so0, so1)

    def cin(e):
        s = e % 2
        return pltpu.make_async_copy(
            ctx_hbm.at[labels[e]], cbufs[s], cgs[s])

    def sin(e):
        s = e % 2
        return pltpu.make_async_copy(
            suf_hbm.at[labels[e]], sbufs[s], sgs[s])

    def cout(e):
        s = e % 2
        return pltpu.make_async_copy(cbufs[s], gctx_hbm.at[base + e], cos[s])

    def sout(e):
        s = e % 2
        return pltpu.make_async_copy(
            sbufs[s], gsuf_hbm.at[base + e], sos[s])

    cin(0).start()
    sin(0).start()
    cin(1).start()
    sin(1).start()

    for e in range(BPW):
        cin(e).wait()
        cout(e).start()
        sin(e).wait()
        sout(e).start()
        if e + 2 < BPW:
            sout(e).wait()
            sin(e + 2).start()
        else:
            sout(e).wait()
        if e + 2 < BPW:
            cout(e).wait()
            cin(e + 2).start()
        else:
            cout(e).wait()


B_BLK = 16


def _concat_tc(pre_ref, gctx_ref, gsuf_ref, out_ref):
    out_ref[:, 0:1, :] = jnp.broadcast_to(pre_ref[...], (B_BLK, 1, CTX_DIM))
    out_ref[:, 1:1 + N_CTX, :] = gctx_ref[...]
    out_ref[:, 1 + N_CTX:SEQ_LEN, :] = gsuf_ref[...]


_assemble_tc = pl.pallas_call(
    _concat_tc,
    out_shape=jax.ShapeDtypeStruct((BATCH, SEQ_LEN, CTX_DIM), jnp.float32),
    grid=(BATCH // B_BLK,),
    in_specs=[
        pl.BlockSpec((1, 1, CTX_DIM), lambda i: (0, 0, 0)),
        pl.BlockSpec((B_BLK, N_CTX, CTX_DIM), lambda i: (i, 0, 0)),
        pl.BlockSpec((B_BLK, N_SUF, CTX_DIM), lambda i: (i, 0, 0)),
    ],
    out_specs=pl.BlockSpec((B_BLK, SEQ_LEN, CTX_DIM), lambda i: (i, 0, 0)),
)


@jax.jit
def kernel(label, cls_ctx, token_prefix, token_suffix):
    gctx, gsuf = _gather_sc(label.astype(jnp.int32), cls_ctx, token_suffix)
    return _assemble_tc(token_prefix, gctx, gsuf)
